# baseline (speedup 1.0000x reference)
import functools

import numpy as np

import jax
import jax.numpy as jnp
from jax import lax
from jax.experimental import pallas as pl
from jax.experimental.pallas import tpu as pltpu
from jax.experimental.pallas import tpu_sc as plsc

N_NODES = 10000
D = 128
NP = 10112
NPQ = 10112
E_TOT = 330000
NC = 2
NS = 16
NW = NC * NS
EB = 80
E_PAD = 332800
EW = E_PAD // NW
NBLK = EW // EB
EB2 = 1040
NB2 = EW // EB2
E_ALL = E_PAD + EB2


def _proj_body(x_ref, wl_ref, bl_ref, wr_ref, br_ref, xl_ref, xr_ref):
    x = x_ref[...]
    dn = (((1,), (1,)), ((), ()))
    xl_ref[...] = lax.dot_general(x, wl_ref[...], dn,
                                  preferred_element_type=jnp.float32) + bl_ref[...]
    xr_ref[...] = lax.dot_general(x, wr_ref[...], dn,
                                  preferred_element_type=jnp.float32) + br_ref[...]


def _proj(x_pad, W_l, b_l, W_r, b_r):
    blk = NP // 4
    return pl.pallas_call(
        _proj_body,
        grid=(NP // blk,),
        in_specs=[
            pl.BlockSpec((blk, D), lambda i: (i, 0)),
            pl.BlockSpec((D, D), lambda i: (0, 0)),
            pl.BlockSpec((1, D), lambda i: (0, 0)),
            pl.BlockSpec((D, D), lambda i: (0, 0)),
            pl.BlockSpec((1, D), lambda i: (0, 0)),
        ],
        out_specs=[
            pl.BlockSpec((blk, D), lambda i: (i, 0)),
            pl.BlockSpec((blk, D), lambda i: (i, 0)),
        ],
        out_shape=[
            jax.ShapeDtypeStruct((NP, D), jnp.float32),
            jax.ShapeDtypeStruct((NP, D), jnp.float32),
        ],
    )(x_pad, W_l, b_l, W_r, b_r)


_GDN = lax.GatherDimensionNumbers(
    offset_dims=(), collapsed_slice_dims=(0,), start_index_map=(0,))


def _shuf(v, perm):
    return lax.gather(v, perm[:, None], _GDN, slice_sizes=(1,),
                      mode=lax.GatherScatterMode.PROMISE_IN_BOUNDS)


def _hsum16(v):
    lane = lax.iota(jnp.int32, 16)
    for k in (8, 4, 2, 1):
        v = v + _shuf(v, lane ^ k)
    return v


def _sc_edge_body(xl_hbm, xr_hbm, src_hbm, dst_hbm, att_hbm, zeros_hbm,
                  acc_out, ev_out, sidx, didx, didxs, xlb, xrb, evb, attv,
                  acc_sp, semi0, semi1, semg0, semg1, sems0, sems1,
                  semv0, semv1):
    semi = (semi0, semi1)
    semg = (semg0, semg1)
    sems = (sems0, sems1)
    semv = (semv0, semv1)
    c = lax.axis_index("c")
    s = lax.axis_index("s")
    wid = c * NS + s
    rows_per = NP // NS
    r0 = s * rows_per
    pltpu.sync_copy(zeros_hbm.at[pl.ds(r0, rows_per)], acc_sp.at[pl.ds(r0, rows_per)])
    pltpu.sync_copy(att_hbm, attv)
    lane = lax.iota(jnp.int32, 16)
    plsc.subcore_barrier()

    base = wid * EW

    def idx_copy(b, p):
        off = base + b * EB
        c1 = pltpu.async_copy(src_hbm.at[pl.ds(off, EB)], sidx.at[p], semi[p])
        c2 = pltpu.async_copy(dst_hbm.at[pl.ds(off, EB)], didx.at[p], semi[p])
        return c1, c2

    def idx_wait(p):
        pltpu.make_async_copy(src_hbm.at[pl.ds(0, EB)], sidx.at[p], semi[p]).wait()
        pltpu.make_async_copy(dst_hbm.at[pl.ds(0, EB)], didx.at[p], semi[p]).wait()

    def gather_issue(p):
        pltpu.async_copy(xl_hbm.at[sidx.at[p]], xlb.at[p], semg[p])
        pltpu.async_copy(xr_hbm.at[didx.at[p]], xrb.at[p], semg[p])

    def gather_wait(p):
        pltpu.make_async_copy(xl_hbm.at[sidx.at[p]], xlb.at[p], semg[p]).wait()
        pltpu.make_async_copy(xr_hbm.at[didx.at[p]], xrb.at[p], semg[p]).wait()

    att_r = [attv[pl.ds(ci * 16, 16)] for ci in range(D // 16)]
    NCH = D // 16

    def compute_block(b, p):
        def group(g, carry2):
            e0 = g * 16
            didxg = didx[p, pl.ds(e0, 16)]
            didxs[p, pl.ds(e0, 16)] = didxg
            ev16 = jnp.zeros((16,), jnp.float32)
            for j in range(16):
                e = e0 + j
                dot0 = None
                dot1 = None
                for ci in range(0, NCH, 2):
                    z = xlb[p, e, pl.ds(ci * 16, 16)] + xrb[p, e, pl.ds(ci * 16, 16)]
                    t = att_r[ci] * jnp.maximum(z, 0.2 * z)
                    dot0 = t if dot0 is None else dot0 + t
                    z = xlb[p, e, pl.ds(ci * 16 + 16, 16)] + xrb[p, e, pl.ds(ci * 16 + 16, 16)]
                    t = att_r[ci + 1] * jnp.maximum(z, 0.2 * z)
                    dot1 = t if dot1 is None else dot1 + t
                ev = jnp.exp(_hsum16(dot0 + dot1))
                for ci in range(NCH):
                    xlb[p, e, pl.ds(ci * 16, 16)] = ev * xlb[p, e, pl.ds(ci * 16, 16)]
                ev16 = jnp.where(lane == j, ev, ev16)
            evb[pl.ds(p * EB + e0, 16)] = ev16
            return carry2

        lax.fori_loop(0, EB // 16, group, 0)
        pltpu.async_copy(xlb.at[p], acc_sp.at[didxs.at[p]], sems[p], add=True)
        pltpu.async_copy(evb.at[pl.ds(p * EB, EB)],
                         ev_out.at[pl.ds(base + b * EB, EB)], semv[p])

    def scatter_wait(p):
        pltpu.make_async_copy(xlb.at[p], acc_sp.at[didxs.at[p]], sems[p]).wait()
        pltpu.make_async_copy(evb.at[pl.ds(p * EB, EB)],
                              ev_out.at[pl.ds(0, EB)], semv[p]).wait()

    c1, c2 = idx_copy(0, 0)
    c1.wait()
    c2.wait()
    idx_copy(1, 1)
    gather_issue(0)

    for p in (0, 1):
        gather_wait(p)
        idx_wait(1 - p)
        gather_issue(1 - p)
        compute_block(p, p)
        idx_copy(p + 2, p)

    def pair(i, carry):
        b = i * 2
        for p in (0, 1):
            gather_wait(p)
            idx_wait(1 - p)
            scatter_wait(1 - p)
            gather_issue(1 - p)
            compute_block(b + p, p)
            idx_copy(b + p + 2, p)
        return carry

    lax.fori_loop(1, NBLK // 2, pair, 0)
    gather_wait(0)
    idx_wait(1)
    scatter_wait(0)
    scatter_wait(1)
    plsc.subcore_barrier()
    pltpu.sync_copy(acc_sp.at[pl.ds(r0, rows_per)],
                    acc_out.at[c, pl.ds(r0, rows_per)])


def _sc_edge(xl_pad, xr_pad, src, dst, att, zeros):
    mesh = plsc.VectorSubcoreMesh(core_axis_name="c", subcore_axis_name="s")
    f = functools.partial(
        pl.kernel,
        mesh=mesh,
        out_type=[
            jax.ShapeDtypeStruct((NC, NP, D), jnp.float32),
            jax.ShapeDtypeStruct((E_ALL,), jnp.float32),
        ],
        scratch_types=[
            pltpu.VMEM((2, EB), jnp.int32),
            pltpu.VMEM((2, EB), jnp.int32),
            pltpu.VMEM((2, EB), jnp.int32),
            pltpu.VMEM((2, EB, D), jnp.float32),
            pltpu.VMEM((2, EB, D), jnp.float32),
            pltpu.VMEM((2 * EB,), jnp.float32),
            pltpu.VMEM((D,), jnp.float32),
            pltpu.VMEM_SHARED((NP, D), jnp.float32),
            pltpu.SemaphoreType.DMA,
            pltpu.SemaphoreType.DMA,
            pltpu.SemaphoreType.DMA,
            pltpu.SemaphoreType.DMA,
            pltpu.SemaphoreType.DMA,
            pltpu.SemaphoreType.DMA,
            pltpu.SemaphoreType.DMA,
            pltpu.SemaphoreType.DMA,
        ],
    )(_sc_edge_body)
    return f(xl_pad, xr_pad, src, dst, att, zeros)


def _sc_den_body(dst_hbm, ev_hbm, den_out, didx2, evb2, den_t, semi0, semi1):
    semi = (semi0, semi1)
    c = lax.axis_index("c")
    s = lax.axis_index("s")
    wid = c * NS + s
    base = wid * EW
    lane = lax.iota(jnp.int32, 16)
    zv = jnp.zeros((16,), jnp.float32)

    BK = NPQ + 16

    def zinit(w, carry):
        for bk in range(4):
            den_t[pl.ds(bk * BK + w * 16, 16)] = zv
        return carry

    lax.fori_loop(0, BK // 16, zinit, 0)

    def idx_copy(b, p):
        off = base + b * EB2
        c1 = pltpu.async_copy(dst_hbm.at[pl.ds(off, EB2)],
                              didx2.at[pl.ds(p * EB2, EB2)], semi[p])
        c2 = pltpu.async_copy(ev_hbm.at[pl.ds(off, EB2)],
                              evb2.at[pl.ds(p * EB2, EB2)], semi[p])
        return c1, c2

    def idx_wait(p):
        pltpu.make_async_copy(dst_hbm.at[pl.ds(0, EB2)],
                              didx2.at[pl.ds(p * EB2, EB2)], semi[p]).wait()
        pltpu.make_async_copy(ev_hbm.at[pl.ds(0, EB2)],
                              evb2.at[pl.ds(p * EB2, EB2)], semi[p]).wait()

    idx_copy(0, 0)

    def process(b, p):
        idx_wait(p)
        idx_copy(b + 1, 1 - p)

        def grp(g, carry):
            e0 = p * EB2 + g * 16
            didxg = didx2[pl.ds(e0, 16)]
            evg = evb2[pl.ds(e0, 16)]
            for j in range(16):
                bk = j % 4
                di = didxg[j]
                evs = jnp.full((16,), evg[j], jnp.float32)
                dbase = lax.bitwise_and(di, -16) + (bk * BK)
                msk = lane == lax.bitwise_and(di, 15)
                cur = den_t[pl.ds(dbase, 16)]
                den_t[pl.ds(dbase, 16)] = cur + jnp.where(msk, evs, 0.0)
            return carry

        lax.fori_loop(0, EB2 // 16, grp, 0)

    def pair(i, carry):
        b = i * 2
        process(b, 0)
        process(b + 1, 1)
        return carry

    lax.fori_loop(0, NB2 // 2, pair, 0)
    idx_wait(0)

    def comb(w, carry):
        w16 = w * 16
        den_t[pl.ds(w16, 16)] = \
            (den_t[pl.ds(w16, 16)] + den_t[pl.ds(BK + w16, 16)]) + \
            (den_t[pl.ds(2 * BK + w16, 16)] + den_t[pl.ds(3 * BK + w16, 16)])
        return carry

    lax.fori_loop(0, NPQ // 16, comb, 0)
    pltpu.sync_copy(den_t.at[pl.ds(0, NPQ)], den_out.at[wid])


def _sc_den(dst, ev):
    mesh = plsc.VectorSubcoreMesh(core_axis_name="c", subcore_axis_name="s")
    f = functools.partial(
        pl.kernel,
        mesh=mesh,
        out_type=jax.ShapeDtypeStruct((NW, NPQ), jnp.float32),
        scratch_types=[
            pltpu.VMEM((2 * EB2,), jnp.int32),
            pltpu.VMEM((2 * EB2,), jnp.float32),
            pltpu.VMEM((4 * (NPQ + 16),), jnp.float32),
            pltpu.SemaphoreType.DMA,
            pltpu.SemaphoreType.DMA,
        ],
    )(_sc_den_body)
    return f(dst, ev)


def _densum_body(den_ref, out_ref):
    out_ref[...] = jnp.sum(den_ref[...], axis=0, keepdims=True)


def _densum(den):
    return pl.pallas_call(
        _densum_body,
        out_shape=jax.ShapeDtypeStruct((1, NPQ), jnp.float32),
    )(den)


def _mlp_body(acc_ref, den_ref, cb_ref, lw_ref, lb_ref, l2w_ref, l2b_ref,
              out_ref):
    unnorm = acc_ref[0] + acc_ref[1]
    denom = den_ref[...]
    h = unnorm / denom + cb_ref[...]
    h = jnp.maximum(h, 0.0)
    dn = (((1,), (1,)), ((), ()))
    h = lax.dot_general(h, lw_ref[...], dn,
                        preferred_element_type=jnp.float32) + lb_ref[...]
    h = jnp.maximum(h, 0.0)
    h2 = lax.dot_general(h, l2w_ref[...], dn, preferred_element_type=jnp.float32)
    out_ref[...] = h2[:, :1] + l2b_ref[0]


def _mlp(acc, den_col, conv_bias, lin_W, lin_b, lin2_W, lin2_b):
    blk = 2000
    return pl.pallas_call(
        _mlp_body,
        grid=(N_NODES // blk,),
        in_specs=[
            pl.BlockSpec((NC, blk, D), lambda i: (0, i, 0)),
            pl.BlockSpec((blk, 1), lambda i: (i, 0)),
            pl.BlockSpec((1, D), lambda i: (0, 0)),
            pl.BlockSpec((D, D), lambda i: (0, 0)),
            pl.BlockSpec((1, D), lambda i: (0, 0)),
            pl.BlockSpec((D, D), lambda i: (0, 0)),
            pl.BlockSpec(memory_space=pltpu.SMEM),
        ],
        out_specs=pl.BlockSpec((blk, 1), lambda i: (i, 0)),
        out_shape=jax.ShapeDtypeStruct((N_NODES, 1), jnp.float32),
    )(acc, den_col, conv_bias, lin_W, lin_b, lin2_W, lin2_b)


def kernel(x, edge_index, W_l, b_l, W_r, b_r, att, conv_bias,
           lin_W, lin_b, lin2_W, lin2_b):
    x_pad = jnp.concatenate(
        [x, jnp.zeros((NP - N_NODES, D), jnp.float32)], axis=0)
    xl_pad, xr_pad = _proj(x_pad, W_l, b_l.reshape(1, D), W_r, b_r.reshape(1, D))

    loops = jnp.arange(N_NODES, dtype=jnp.int32)
    pad = jnp.full((E_ALL - E_TOT,), N_NODES, dtype=jnp.int32)
    src = jnp.concatenate([edge_index[0].astype(jnp.int32), loops, pad])
    dst = jnp.concatenate([edge_index[1].astype(jnp.int32), loops, pad])

    zeros = jnp.zeros((NP, D), jnp.float32)
    acc, ev = _sc_edge(xl_pad, xr_pad, src, dst, att, zeros)
    den = _sc_den(dst, ev)
    den_col = _densum(den).reshape(NPQ, 1)

    lin2_W_pad = jnp.zeros((D, D), jnp.float32).at[:1].set(lin2_W)
    return _mlp(acc, den_col, conv_bias.reshape(1, D), lin_W,
                lin_b.reshape(1, D), lin2_W_pad, lin2_b)


# speedup vs baseline: 1.0018x; 1.0018x over previous
import functools

import jax
import jax.numpy as jnp
from jax import lax
from jax.experimental import pallas as pl
from jax.experimental.pallas import tpu as pltpu
from jax.experimental.pallas import tpu_sc as plsc

N_NODES = 10000
D = 128
NP = 10112
NPQ = 10112
E_TOT = 330000
NC = 2
NS = 16
NW = NC * NS
EB = 80
E_PAD = 332800
EW = E_PAD // NW
NBLK = EW // EB
EB2 = 1040
NB2 = EW // EB2
E_ALL = E_PAD + EB2


def _proj_body(x_ref, wl_ref, bl_ref, wr_ref, br_ref, xl_ref, xr_ref):
    x = x_ref[...]
    dn = (((1,), (1,)), ((), ()))
    xl_ref[...] = lax.dot_general(x, wl_ref[...], dn,
                                  preferred_element_type=jnp.float32) + bl_ref[...]
    xr_ref[...] = lax.dot_general(x, wr_ref[...], dn,
                                  preferred_element_type=jnp.float32) + br_ref[...]


def _proj(x_pad, W_l, b_l, W_r, b_r):
    blk = NP // 4
    return pl.pallas_call(
        _proj_body,
        grid=(NP // blk,),
        in_specs=[
            pl.BlockSpec((blk, D), lambda i: (i, 0)),
            pl.BlockSpec((D, D), lambda i: (0, 0)),
            pl.BlockSpec((1, D), lambda i: (0, 0)),
            pl.BlockSpec((D, D), lambda i: (0, 0)),
            pl.BlockSpec((1, D), lambda i: (0, 0)),
        ],
        out_specs=[
            pl.BlockSpec((blk, D), lambda i: (i, 0)),
            pl.BlockSpec((blk, D), lambda i: (i, 0)),
        ],
        out_shape=[
            jax.ShapeDtypeStruct((NP, D), jnp.float32),
            jax.ShapeDtypeStruct((NP, D), jnp.float32),
        ],
    )(x_pad, W_l, b_l, W_r, b_r)


_GDN = lax.GatherDimensionNumbers(
    offset_dims=(), collapsed_slice_dims=(0,), start_index_map=(0,))


def _shuf(v, perm):
    return lax.gather(v, perm[:, None], _GDN, slice_sizes=(1,),
                      mode=lax.GatherScatterMode.PROMISE_IN_BOUNDS)


def _hsum16(v):
    lane = lax.iota(jnp.int32, 16)
    for k in (8, 4, 2, 1):
        v = v + _shuf(v, lane ^ k)
    return v


def _sc_edge_body(xl_hbm, xr_hbm, src_hbm, dst_hbm, att_hbm, zeros_hbm,
                  acc_out, ev_out, sidx, didx, didxs, xlb, xrb, evb, attv,
                  acc_sp, semi0, semi1, semg0, semg1, sems0, sems1,
                  semv0, semv1):
    semi = (semi0, semi1)
    semg = (semg0, semg1)
    sems = (sems0, sems1)
    semv = (semv0, semv1)
    c = lax.axis_index("c")
    s = lax.axis_index("s")
    wid = c * NS + s
    rows_per = NP // NS
    r0 = s * rows_per
    pltpu.sync_copy(zeros_hbm.at[pl.ds(r0, rows_per)], acc_sp.at[pl.ds(r0, rows_per)])
    pltpu.sync_copy(att_hbm, attv)
    lane = lax.iota(jnp.int32, 16)
    plsc.subcore_barrier()

    base = wid * EW

    def idx_copy(b, p):
        off = base + b * EB
        c1 = pltpu.async_copy(src_hbm.at[pl.ds(off, EB)], sidx.at[p], semi[p])
        c2 = pltpu.async_copy(dst_hbm.at[pl.ds(off, EB)], didx.at[p], semi[p])
        return c1, c2

    def idx_wait(p):
        pltpu.make_async_copy(src_hbm.at[pl.ds(0, EB)], sidx.at[p], semi[p]).wait()
        pltpu.make_async_copy(dst_hbm.at[pl.ds(0, EB)], didx.at[p], semi[p]).wait()

    def gather_issue(p):
        pltpu.async_copy(xl_hbm.at[sidx.at[p]], xlb.at[p], semg[p])
        pltpu.async_copy(xr_hbm.at[didx.at[p]], xrb.at[p], semg[p])

    def gather_wait(p):
        pltpu.make_async_copy(xl_hbm.at[sidx.at[p]], xlb.at[p], semg[p]).wait()
        pltpu.make_async_copy(xr_hbm.at[didx.at[p]], xrb.at[p], semg[p]).wait()

    att_r = [attv[pl.ds(ci * 16, 16)] for ci in range(D // 16)]
    NCH = D // 16

    def compute_block(b, p):
        def group(g, carry2):
            e0 = g * 16
            didxg = didx[p, pl.ds(e0, 16)]
            didxs[p, pl.ds(e0, 16)] = didxg
            ev16 = jnp.zeros((16,), jnp.float32)
            for j in range(16):
                e = e0 + j
                dot0 = None
                dot1 = None
                for ci in range(0, NCH, 2):
                    z = xlb[p, e, pl.ds(ci * 16, 16)] + xrb[p, e, pl.ds(ci * 16, 16)]
                    t = att_r[ci] * jnp.maximum(z, 0.2 * z)
                    dot0 = t if dot0 is None else dot0 + t
                    z = xlb[p, e, pl.ds(ci * 16 + 16, 16)] + xrb[p, e, pl.ds(ci * 16 + 16, 16)]
                    t = att_r[ci + 1] * jnp.maximum(z, 0.2 * z)
                    dot1 = t if dot1 is None else dot1 + t
                ev = jnp.exp(_hsum16(dot0 + dot1))
                for ci in range(NCH):
                    xlb[p, e, pl.ds(ci * 16, 16)] = ev * xlb[p, e, pl.ds(ci * 16, 16)]
                ev16 = jnp.where(lane == j, ev, ev16)
            evb[pl.ds(p * EB + e0, 16)] = ev16
            return carry2

        lax.fori_loop(0, EB // 16, group, 0)
        pltpu.async_copy(xlb.at[p], acc_sp.at[didxs.at[p]], sems[p], add=True)
        pltpu.async_copy(evb.at[pl.ds(p * EB, EB)],
                         ev_out.at[pl.ds(base + b * EB, EB)], semv[p])

    def scatter_wait(p):
        pltpu.make_async_copy(xlb.at[p], acc_sp.at[didxs.at[p]], sems[p]).wait()
        pltpu.make_async_copy(evb.at[pl.ds(p * EB, EB)],
                              ev_out.at[pl.ds(0, EB)], semv[p]).wait()

    c1, c2 = idx_copy(0, 0)
    c1.wait()
    c2.wait()
    idx_copy(1, 1)
    gather_issue(0)

    for p in (0, 1):
        gather_wait(p)
        idx_wait(1 - p)
        if p == 1:
            scatter_wait(0)
        gather_issue(1 - p)
        compute_block(p, p)
        idx_copy(p + 2, p)

    def pair(i, carry):
        b = i * 2
        for p in (0, 1):
            gather_wait(p)
            idx_wait(1 - p)
            scatter_wait(1 - p)
            gather_issue(1 - p)
            compute_block(b + p, p)
            idx_copy(b + p + 2, p)
        return carry

    lax.fori_loop(1, NBLK // 2, pair, 0)
    gather_wait(0)
    idx_wait(1)
    scatter_wait(1)
    plsc.subcore_barrier()
    pltpu.sync_copy(acc_sp.at[pl.ds(r0, rows_per)],
                    acc_out.at[c, pl.ds(r0, rows_per)])


def _sc_edge(xl_pad, xr_pad, src, dst, att, zeros):
    mesh = plsc.VectorSubcoreMesh(core_axis_name="c", subcore_axis_name="s")
    f = functools.partial(
        pl.kernel,
        mesh=mesh,
        out_type=[
            jax.ShapeDtypeStruct((NC, NP, D), jnp.float32),
            jax.ShapeDtypeStruct((E_ALL,), jnp.float32),
        ],
        scratch_types=[
            pltpu.VMEM((2, EB), jnp.int32),
            pltpu.VMEM((2, EB), jnp.int32),
            pltpu.VMEM((2, EB), jnp.int32),
            pltpu.VMEM((2, EB, D), jnp.float32),
            pltpu.VMEM((2, EB, D), jnp.float32),
            pltpu.VMEM((2 * EB,), jnp.float32),
            pltpu.VMEM((D,), jnp.float32),
            pltpu.VMEM_SHARED((NP, D), jnp.float32),
            pltpu.SemaphoreType.DMA,
            pltpu.SemaphoreType.DMA,
            pltpu.SemaphoreType.DMA,
            pltpu.SemaphoreType.DMA,
            pltpu.SemaphoreType.DMA,
            pltpu.SemaphoreType.DMA,
            pltpu.SemaphoreType.DMA,
            pltpu.SemaphoreType.DMA,
        ],
    )(_sc_edge_body)
    return f(xl_pad, xr_pad, src, dst, att, zeros)


def _sc_den_body(dst_hbm, ev_hbm, den_out, didx2, evb2, den_t, semi0, semi1):
    semi = (semi0, semi1)
    c = lax.axis_index("c")
    s = lax.axis_index("s")
    wid = c * NS + s
    base = wid * EW
    lane = lax.iota(jnp.int32, 16)
    zv = jnp.zeros((16,), jnp.float32)

    BK = NPQ + 16

    def zinit(w, carry):
        for bk in range(4):
            den_t[pl.ds(bk * BK + w * 16, 16)] = zv
        return carry

    lax.fori_loop(0, BK // 16, zinit, 0)

    def idx_copy(b, p):
        off = base + b * EB2
        c1 = pltpu.async_copy(dst_hbm.at[pl.ds(off, EB2)],
                              didx2.at[pl.ds(p * EB2, EB2)], semi[p])
        c2 = pltpu.async_copy(ev_hbm.at[pl.ds(off, EB2)],
                              evb2.at[pl.ds(p * EB2, EB2)], semi[p])
        return c1, c2

    def idx_wait(p):
        pltpu.make_async_copy(dst_hbm.at[pl.ds(0, EB2)],
                              didx2.at[pl.ds(p * EB2, EB2)], semi[p]).wait()
        pltpu.make_async_copy(ev_hbm.at[pl.ds(0, EB2)],
                              evb2.at[pl.ds(p * EB2, EB2)], semi[p]).wait()

    idx_copy(0, 0)

    def process(b, p):
        idx_wait(p)
        idx_copy(b + 1, 1 - p)

        def grp(g, carry):
            e0 = p * EB2 + g * 16
            didxg = didx2[pl.ds(e0, 16)]
            evg = evb2[pl.ds(e0, 16)]
            for j in range(16):
                bk = j % 4
                di = didxg[j]
                evs = jnp.full((16,), evg[j], jnp.float32)
                dbase = lax.bitwise_and(di, -16) + (bk * BK)
                msk = lane == lax.bitwise_and(di, 15)
                cur = den_t[pl.ds(dbase, 16)]
                den_t[pl.ds(dbase, 16)] = cur + jnp.where(msk, evs, 0.0)
            return carry

        lax.fori_loop(0, EB2 // 16, grp, 0)

    def pair(i, carry):
        b = i * 2
        process(b, 0)
        process(b + 1, 1)
        return carry

    lax.fori_loop(0, NB2 // 2, pair, 0)
    idx_wait(0)

    def comb(w, carry):
        w16 = w * 16
        den_t[pl.ds(w16, 16)] = \
            (den_t[pl.ds(w16, 16)] + den_t[pl.ds(BK + w16, 16)]) + \
            (den_t[pl.ds(2 * BK + w16, 16)] + den_t[pl.ds(3 * BK + w16, 16)])
        return carry

    lax.fori_loop(0, NPQ // 16, comb, 0)
    pltpu.sync_copy(den_t.at[pl.ds(0, NPQ)], den_out.at[wid])


def _sc_den(dst, ev):
    mesh = plsc.VectorSubcoreMesh(core_axis_name="c", subcore_axis_name="s")
    f = functools.partial(
        pl.kernel,
        mesh=mesh,
        out_type=jax.ShapeDtypeStruct((NW, NPQ), jnp.float32),
        scratch_types=[
            pltpu.VMEM((2 * EB2,), jnp.int32),
            pltpu.VMEM((2 * EB2,), jnp.float32),
            pltpu.VMEM((4 * (NPQ + 16),), jnp.float32),
            pltpu.SemaphoreType.DMA,
            pltpu.SemaphoreType.DMA,
        ],
    )(_sc_den_body)
    return f(dst, ev)


def _densum_body(den_ref, out_ref):
    out_ref[...] = jnp.sum(den_ref[...], axis=0, keepdims=True)


def _densum(den):
    return pl.pallas_call(
        _densum_body,
        out_shape=jax.ShapeDtypeStruct((1, NPQ), jnp.float32),
    )(den)


def _mlp_body(acc_ref, den_ref, cb_ref, lw_ref, lb_ref, l2w_ref, l2b_ref,
              out_ref):
    unnorm = acc_ref[0] + acc_ref[1]
    denom = den_ref[...]
    h = unnorm / denom + cb_ref[...]
    h = jnp.maximum(h, 0.0)
    dn = (((1,), (1,)), ((), ()))
    h = lax.dot_general(h, lw_ref[...], dn,
                        preferred_element_type=jnp.float32) + lb_ref[...]
    h = jnp.maximum(h, 0.0)
    h2 = lax.dot_general(h, l2w_ref[...], dn, preferred_element_type=jnp.float32)
    out_ref[...] = h2[:, :1] + l2b_ref[0]


def _mlp(acc, den_col, conv_bias, lin_W, lin_b, lin2_W, lin2_b):
    blk = 2000
    return pl.pallas_call(
        _mlp_body,
        grid=(N_NODES // blk,),
        in_specs=[
            pl.BlockSpec((NC, blk, D), lambda i: (0, i, 0)),
            pl.BlockSpec((blk, 1), lambda i: (i, 0)),
            pl.BlockSpec((1, D), lambda i: (0, 0)),
            pl.BlockSpec((D, D), lambda i: (0, 0)),
            pl.BlockSpec((1, D), lambda i: (0, 0)),
            pl.BlockSpec((D, D), lambda i: (0, 0)),
            pl.BlockSpec(memory_space=pltpu.SMEM),
        ],
        out_specs=pl.BlockSpec((blk, 1), lambda i: (i, 0)),
        out_shape=jax.ShapeDtypeStruct((N_NODES, 1), jnp.float32),
    )(acc, den_col, conv_bias, lin_W, lin_b, lin2_W, lin2_b)


def kernel(x, edge_index, W_l, b_l, W_r, b_r, att, conv_bias,
           lin_W, lin_b, lin2_W, lin2_b):
    x_pad = jnp.concatenate(
        [x, jnp.zeros((NP - N_NODES, D), jnp.float32)], axis=0)
    xl_pad, xr_pad = _proj(x_pad, W_l, b_l.reshape(1, D), W_r, b_r.reshape(1, D))

    loops = jnp.arange(N_NODES, dtype=jnp.int32)
    pad = jnp.full((E_ALL - E_TOT,), N_NODES, dtype=jnp.int32)
    src = jnp.concatenate([edge_index[0].astype(jnp.int32), loops, pad])
    dst = jnp.concatenate([edge_index[1].astype(jnp.int32), loops, pad])

    zeros = jnp.zeros((NP, D), jnp.float32)
    acc, ev = _sc_edge(xl_pad, xr_pad, src, dst, att, zeros)
    den = _sc_den(dst, ev)
    den_col = _densum(den).reshape(NPQ, 1)

    lin2_W_pad = jnp.zeros((D, D), jnp.float32).at[:1].set(lin2_W)
    return _mlp(acc, den_col, conv_bias.reshape(1, D), lin_W,
                lin_b.reshape(1, D), lin2_W_pad, lin2_b)
